# baseline (speedup 1.0000x reference)
import functools

import jax
import jax.numpy as jnp
from jax import lax
from jax.experimental import pallas as pl
from jax.experimental.pallas import tpu as pltpu
from jax.experimental.pallas import tpu_sc as plsc

def _f32_bits(x):
    return lax.bitcast_convert_type(
        x.astype(jnp.bfloat16).astype(jnp.float32), jnp.int32)


def _unpack_lo(p):
    return lax.bitcast_convert_type(lax.shift_left(p, 16), jnp.float32)


def _unpack_hi(p):
    return lax.bitcast_convert_type(p & jnp.int32(-65536), jnp.float32)


def _unpack(p):
    return jnp.concatenate([_unpack_lo(p), _unpack_hi(p)], axis=1)


N_NODES = 10000
N_EDGES = 160000
HIDDEN = 256
HEADS = 16
ADIM = 16
HD = HEADS * ADIM

NC = 2
NS = 16
N_PAD = 10240


def _tc_qkv(x, wt, b3):
    n = x.shape[0]
    bn = 1000
    gi = n // bn

    def body(x_ref, wt_ref, b_ref, o_ref):
        q = (
            jnp.dot(x_ref[...], wt_ref[...], preferred_element_type=jnp.float32)
            + b_ref[0]
        )
        a = _f32_bits(q[:, :128])
        b = _f32_bits(q[:, 128:])
        o_ref[...] = b | lax.shift_right_logical(a, 16)

    return pl.pallas_call(
        body,
        grid=(3, gi),
        in_specs=[
            pl.BlockSpec((bn, HIDDEN), lambda j, i: (i, 0)),
            pl.BlockSpec((HIDDEN, HIDDEN), lambda j, i: (0, j)),
            pl.BlockSpec((1, 1, HIDDEN), lambda j, i: (j, 0, 0)),
        ],
        out_specs=pl.BlockSpec((bn, HIDDEN // 2), lambda j, i: (j * gi + i, 0)),
        out_shape=jax.ShapeDtypeStruct((3 * n, HIDDEN // 2), jnp.int32),
    )(x, wt, b3)


def _sc_gather(table, gidx):
    rows = gidx.shape[0]
    width = table.shape[1]
    dt = table.dtype
    nw = NC * NS
    per_w = rows // nw
    ch = 120
    nch = per_w // ch

    mesh = plsc.VectorSubcoreMesh(core_axis_name="c", subcore_axis_name="s")

    @functools.partial(
        pl.kernel,
        out_type=jax.ShapeDtypeStruct((rows, width), dt),
        mesh=mesh,
        scratch_types=[
            pltpu.VMEM((per_w,), jnp.int32),
            pltpu.VMEM((ch, width), dt),
            pltpu.VMEM((ch, width), dt),
            pltpu.SemaphoreType.DMA,
            pltpu.SemaphoreType.DMA,
            pltpu.SemaphoreType.DMA,
            pltpu.SemaphoreType.DMA,
        ],
    )
    def k(table_hbm, gidx_hbm, out_hbm, idx_all, b0, b1, gs0, gs1, ss0, ss1):
        cid = lax.axis_index("c")
        sid = lax.axis_index("s")
        base = (sid * NC + cid) * per_w

        pltpu.sync_copy(gidx_hbm.at[pl.ds(base, per_w)], idx_all)
        pltpu.async_copy(table_hbm.at[idx_all.at[pl.ds(0, ch)]], b0, gs0)
        pltpu.async_copy(table_hbm.at[idx_all.at[pl.ds(ch, ch)]], b1, gs1)

        @pl.loop(0, (nch - 1) // 2)
        def _(p):
            c0 = 2 * p
            pltpu.make_async_copy(
                table_hbm.at[idx_all.at[pl.ds(0, ch)]], b0, gs0).wait()
            pltpu.async_copy(b0, out_hbm.at[pl.ds(base + c0 * ch, ch)], ss0)
            pltpu.make_async_copy(
                table_hbm.at[idx_all.at[pl.ds(0, ch)]], b1, gs1).wait()
            pltpu.async_copy(b1, out_hbm.at[pl.ds(base + (c0 + 1) * ch, ch)], ss1)

            pltpu.make_async_copy(b0, out_hbm.at[pl.ds(base, ch)], ss0).wait()
            pltpu.async_copy(
                table_hbm.at[idx_all.at[pl.ds((c0 + 2) * ch, ch)]], b0, gs0)
            pltpu.make_async_copy(b1, out_hbm.at[pl.ds(base, ch)], ss1).wait()

            @pl.when(c0 + 3 < nch)
            def _():
                pltpu.async_copy(
                    table_hbm.at[idx_all.at[pl.ds((c0 + 3) * ch, ch)]], b1, gs1)

        pltpu.make_async_copy(
            table_hbm.at[idx_all.at[pl.ds(0, ch)]], b0, gs0).wait()
        pltpu.sync_copy(b0, out_hbm.at[pl.ds(base + (nch - 1) * ch, ch)])

    return k(table, gidx)


def _tc_edge(g, conn, dstm2, ewt, eb2, sa, bbd, r1, r8, sb, esb, eo_prev):
    e = conn.shape[0]
    be = 2000
    gi = esb // be
    off = sb * gi

    def body(qd, ks, vs, cn, dm_r, ewt_r, eb_r, sa_r, bbd_r, r1_r, r8_r,
             eo, ua, ub, w2):
        eh = (
            jnp.dot(cn[...].astype(jnp.bfloat16), ewt_r[...],
                    preferred_element_type=jnp.float32)
            + eb_r[...]
        )
        ew = eh[:, :HD]
        ebb = eh[:, HD:]
        c1 = (_unpack(qd[...]) + _unpack(ks[...])) * ew
        c2 = jnp.sign(c1) * jnp.sqrt(jnp.abs(c1))
        co = jnp.maximum(c2 + ebb, 0.0)
        eo[...] = co
        cob = co.astype(jnp.bfloat16)
        s = jnp.clip(
            jnp.dot(cob, sa_r[...], preferred_element_type=jnp.float32), -5.0, 5.0
        )
        wb = jnp.exp(s).astype(jnp.bfloat16)
        wtile = jnp.dot(
            wb, r8_r[...], preferred_element_type=jnp.float32,
        )
        slot = lax.broadcasted_iota(jnp.int32, (be, 128), 1) // HEADS
        w2[...] = jnp.where(dm_r[...] == slot, wtile, 0.0)
        wrep = jnp.dot(
            wb, r1_r[...], preferred_element_type=jnp.float32,
        )
        u = (_unpack(vs[...])
             + jnp.dot(cob, bbd_r[...], preferred_element_type=jnp.float32)) * wrep
        ua[...] = u[:, :128]
        ub[...] = u[:, 128:]

    in_specs = [
        pl.BlockSpec((be, HIDDEN // 2), lambda i: (i, 0)),
        pl.BlockSpec((be, HIDDEN // 2), lambda i: (i + gi, 0)),
        pl.BlockSpec((be, HIDDEN // 2), lambda i: (i + 2 * gi, 0)),
        pl.BlockSpec((be, HIDDEN), lambda i: (i + off, 0)),
        pl.BlockSpec((be, 1), lambda i: (i + off, 0)),
        pl.BlockSpec((HIDDEN, 2 * HD), lambda i: (0, 0)),
        pl.BlockSpec((1, 2 * HD), lambda i: (0, 0)),
        pl.BlockSpec((HD, HEADS), lambda i: (0, 0)),
        pl.BlockSpec((HD, HD), lambda i: (0, 0)),
        pl.BlockSpec((HEADS, HD), lambda i: (0, 0)),
        pl.BlockSpec((HEADS, 128), lambda i: (0, 0)),
    ]
    args = [g, g, g, conn, dstm2, ewt, eb2, sa, bbd, r1, r8]
    aliases = {}
    if eo_prev is not None:
        in_specs.append(pl.BlockSpec(memory_space=pl.ANY))
        args.append(eo_prev)
        aliases[11] = 0

    def body_wrap(*refs):
        body(*refs[:11], *refs[-4:])

    return pl.pallas_call(
        body_wrap if eo_prev is not None else body,
        grid=(gi,),
        in_specs=in_specs,
        out_specs=[
            pl.BlockSpec((be, HD), lambda i: (i + off, 0)),
            pl.BlockSpec((be, 128), lambda i: (i, 0)),
            pl.BlockSpec((be, 128), lambda i: (i, 0)),
            pl.BlockSpec((be, 128), lambda i: (i, 0)),
        ],
        out_shape=[
            jax.ShapeDtypeStruct((e, HD), jnp.float32),
            jax.ShapeDtypeStruct((esb, 128), jnp.float32),
            jax.ShapeDtypeStruct((esb, 128), jnp.float32),
            jax.ShapeDtypeStruct((esb, 128), jnp.float32),
        ],
        input_output_aliases=aliases,
    )(*args)


def _sc_scatter(ua, ub, w2, dst, dst8):
    e = ua.shape[0]
    n = N_PAD
    nw = n // 8
    per_t = e // NS
    ch = 40
    nch = per_t // ch
    kq = 5
    nq = nch // kq
    rows_t = n // NS
    wrows_t = nw // NS

    mesh = plsc.VectorSubcoreMesh(core_axis_name="c", subcore_axis_name="s")

    @functools.partial(
        pl.kernel,
        out_type=(
            jax.ShapeDtypeStruct((n, 128), jnp.float32),
            jax.ShapeDtypeStruct((n, 128), jnp.float32),
            jax.ShapeDtypeStruct((2, nw, 128), jnp.float32),
        ),
        mesh=mesh,
        scratch_types=[
            pltpu.VMEM_SHARED((n, 128), jnp.float32),
            pltpu.VMEM_SHARED((nw, 128), jnp.float32),
            pltpu.VMEM((kq, ch), jnp.int32),
            pltpu.VMEM((kq, ch), jnp.int32),
            [pltpu.VMEM((ch, 128), jnp.float32) for _ in range(kq)],
            pltpu.SemaphoreType.DMA,
            pltpu.SemaphoreType.DMA,
        ],
    )
    def k(ua_h, ub_h, w2_h, dst3_h, dst83_h, out_a, out_b, out_w, acc_u, acc_w,
          idxq, idx8q, bufs, lsem, asem):
        cid = lax.axis_index("c")
        sid = lax.axis_index("s")

        @pl.loop(0, ch)
        def _(i):
            @pl.loop(0, 128 // 16)
            def _(j):
                bufs[0][i, pl.ds(j * 16, 16)] = jnp.zeros((16,), jnp.float32)

        r0 = sid * rows_t
        w0 = sid * wrows_t

        @pl.loop(0, rows_t // ch)
        def _(b):
            pltpu.sync_copy(bufs[0], acc_u.at[pl.ds(r0 + b * ch, ch)])

        @pl.loop(0, wrows_t // ch)
        def _(b):
            pltpu.sync_copy(bufs[0], acc_w.at[pl.ds(w0 + b * ch, ch)])

        plsc.subcore_barrier()

        base_e = sid * per_t

        def round_phase(src_h, acc, idx_h, idxb, p):
            base_c = p * kq
            pltpu.sync_copy(idx_h.at[sid * nq + p], idxb)
            for b in range(kq):
                pltpu.async_copy(
                    src_h.at[pl.ds(base_e + (base_c + b) * ch, ch)],
                    bufs[b], lsem)
            for b in range(kq):
                pltpu.make_async_copy(
                    src_h.at[pl.ds(base_e, ch)], bufs[b], lsem).wait()
            for b in range(kq):
                pltpu.async_copy(bufs[b], acc.at[idxb.at[b]], asem, add=True)
            for b in range(kq):
                pltpu.make_async_copy(bufs[b], acc.at[idxb.at[0]], asem).wait()

        def main_loop(src_h):
            @pl.loop(0, nq)
            def _(p):
                round_phase(src_h, acc_u, dst3_h, idxq, p)

                @pl.when(lax.rem(p, 2) == cid)
                def _():
                    round_phase(w2_h, acc_w, dst83_h, idx8q, p)

        @pl.when(cid == 0)
        def _():
            main_loop(ua_h)

        @pl.when(cid == 1)
        def _():
            main_loop(ub_h)

        plsc.subcore_barrier()

        @pl.when(cid == 0)
        def _():
            pltpu.sync_copy(acc_u.at[pl.ds(r0, rows_t)], out_a.at[pl.ds(r0, rows_t)])

        @pl.when(cid == 1)
        def _():
            pltpu.sync_copy(acc_u.at[pl.ds(r0, rows_t)], out_b.at[pl.ds(r0, rows_t)])

        pltpu.sync_copy(
            acc_w.at[pl.ds(w0, wrows_t)], out_w.at[cid].at[pl.ds(w0, wrows_t)]
        )

    dst3 = dst.reshape(NS * nq, kq, ch)
    dst83 = dst8.reshape(NS * nq, kq, ch)
    return k(ua, ub, w2, dst3, dst83)


def _tc_node(accs, ws, r1):
    n = N_NODES
    bn = 1000
    gi = n // bn
    na, nw_in = len(accs), len(ws)

    def body(*refs):
        a_refs = refs[:na]
        w_refs = refs[na:na + nw_in]
        r1_r = refs[na + nw_in]
        o = refs[-1]
        half = na // 2
        a = sum(r[...] for r in a_refs[:half])
        b = sum(r[...] for r in a_refs[half:])
        wsum = sum(r[...] for r in w_refs)
        den = (
            jnp.dot(
                wsum, r1_r[...], preferred_element_type=jnp.float32,
                precision=lax.Precision.HIGHEST,
            )
            + 1e-16
        )
        o[...] = jnp.concatenate([a, b], axis=1) / den

    return pl.pallas_call(
        body,
        grid=(gi,),
        in_specs=(
            [pl.BlockSpec((bn, 128), lambda i: (i, 0)) for _ in accs]
            + [pl.BlockSpec((bn, HEADS), lambda i: (i, 0)) for _ in ws]
            + [pl.BlockSpec((HEADS, HD), lambda i: (0, 0))]
        ),
        out_specs=pl.BlockSpec((bn, HD), lambda i: (i, 0)),
        out_shape=jax.ShapeDtypeStruct((n, HD), jnp.float32),
    )(*accs, *ws, r1)


def kernel(x, rrwp_index, rrwp_conn, qkv_weight, qkv_bias, E_weight, E_bias, Aw, Bw):
    n = x.shape[0]
    dst = rrwp_index[0]
    src = rrwp_index[1]

    wt = qkv_weight.T
    b3 = qkv_bias.reshape(3, 1, HIDDEN)
    ewt = E_weight.T.astype(jnp.bfloat16)
    eb2 = E_bias.reshape(1, 2 * HD)
    eye = jnp.eye(HEADS, dtype=jnp.float32)
    sa = (Aw[:, :, 0].T[:, :, None] * eye[:, None, :]).reshape(HD, HEADS)
    sa = sa.astype(jnp.bfloat16)
    bbd = (Bw.transpose(1, 0, 2)[:, :, None, :] * eye[:, None, :, None]).reshape(HD, HD)
    bbd = bbd.astype(jnp.bfloat16)
    r1 = jnp.repeat(eye, ADIM, axis=1).astype(jnp.bfloat16)
    r8 = jnp.tile(eye, (1, 8)).astype(jnp.bfloat16)

    t32 = _tc_qkv(x, wt, b3)
    dstm2 = (dst % 8).reshape(-1, 1)
    dst8 = dst // 8

    nsb = 2
    esb = N_EDGES // nsb
    rows_pad = 241920
    eo = None
    accs, wplanes = [], []
    for sb in range(nsb):
        sl = slice(sb * esb, (sb + 1) * esb)
        gidx = jnp.concatenate([
            dst[sl], src[sl] + n, src[sl] + 2 * n,
            jnp.zeros((rows_pad - 3 * esb,), jnp.int32)])
        g32 = _sc_gather(t32, gidx)
        eo, ua, ub, w2 = _tc_edge(g32, rrwp_conn, dstm2, ewt, eb2, sa, bbd,
                                  r1, r8, sb, esb, eo)
        acc_a, acc_b, acc_w = _sc_scatter(ua, ub, w2, dst[sl], dst8[sl])
        accw2 = acc_w.reshape(2, N_PAD, HEADS)
        accs.extend([acc_a, acc_b])
        wplanes.extend([accw2[0], accw2[1]])
    accs = [accs[0], accs[2], accs[1], accs[3]]
    no = _tc_node(accs, wplanes, jnp.repeat(eye, ADIM, axis=1))
    return no, eo


# speedup vs baseline: 1.1332x; 1.1332x over previous
import functools

import jax
import jax.numpy as jnp
from jax import lax
from jax.experimental import pallas as pl
from jax.experimental.pallas import tpu as pltpu
from jax.experimental.pallas import tpu_sc as plsc

def _f32_bits(x):
    return lax.bitcast_convert_type(
        x.astype(jnp.bfloat16).astype(jnp.float32), jnp.int32)


def _unpack_lo(p):
    return lax.bitcast_convert_type(lax.shift_left(p, 16), jnp.float32)


def _unpack_hi(p):
    return lax.bitcast_convert_type(p & jnp.int32(-65536), jnp.float32)


def _unpack(p):
    return jnp.concatenate([_unpack_lo(p), _unpack_hi(p)], axis=1)


N_NODES = 10000
N_EDGES = 160000
HIDDEN = 256
HEADS = 16
ADIM = 16
HD = HEADS * ADIM

NC = 2
NS = 16
N_PAD = 10240


def _tc_qkv(x, wt, b3):
    n = x.shape[0]
    bn = 1000
    gi = n // bn

    def body(x_ref, wt_ref, b_ref, o_ref):
        q = (
            jnp.dot(x_ref[...], wt_ref[...], preferred_element_type=jnp.float32)
            + b_ref[0]
        )
        a = _f32_bits(q[:, :128])
        b = _f32_bits(q[:, 128:])
        o_ref[...] = b | lax.shift_right_logical(a, 16)

    return pl.pallas_call(
        body,
        grid=(3, gi),
        in_specs=[
            pl.BlockSpec((bn, HIDDEN), lambda j, i: (i, 0)),
            pl.BlockSpec((HIDDEN, HIDDEN), lambda j, i: (0, j)),
            pl.BlockSpec((1, 1, HIDDEN), lambda j, i: (j, 0, 0)),
        ],
        out_specs=pl.BlockSpec((bn, HIDDEN // 2), lambda j, i: (j * gi + i, 0)),
        out_shape=jax.ShapeDtypeStruct((3 * n, HIDDEN // 2), jnp.int32),
    )(x, wt, b3)


def _sc_gather(table, gidx):
    rows = gidx.shape[0]
    width = table.shape[1]
    dt = table.dtype
    nw = NC * NS
    per_w = rows // nw
    ch = 120
    nch = per_w // ch

    mesh = plsc.VectorSubcoreMesh(core_axis_name="c", subcore_axis_name="s")

    @functools.partial(
        pl.kernel,
        out_type=jax.ShapeDtypeStruct((rows, width), dt),
        mesh=mesh,
        scratch_types=[
            pltpu.VMEM((per_w,), jnp.int32),
            pltpu.VMEM((ch, width), dt),
            pltpu.VMEM((ch, width), dt),
            pltpu.VMEM((ch, width), dt),
            pltpu.SemaphoreType.DMA,
            pltpu.SemaphoreType.DMA,
            pltpu.SemaphoreType.DMA,
            pltpu.SemaphoreType.DMA,
            pltpu.SemaphoreType.DMA,
            pltpu.SemaphoreType.DMA,
        ],
    )
    def k(table_hbm, gidx_hbm, out_hbm, idx_all, b0, b1, b2,
          gs0, gs1, gs2, ss0, ss1, ss2):
        cid = lax.axis_index("c")
        sid = lax.axis_index("s")
        base = (sid * NC + cid) * per_w

        bufs = (b0, b1, b2)
        gsems = (gs0, gs1, gs2)
        ssems = (ss0, ss1, ss2)

        def fire_gather(c, b):
            pltpu.async_copy(
                table_hbm.at[idx_all.at[pl.ds(c * ch, ch)]], bufs[b], gsems[b])

        def wait_gather(b):
            pltpu.make_async_copy(
                table_hbm.at[idx_all.at[pl.ds(0, ch)]], bufs[b], gsems[b]).wait()

        def fire_store(c, b):
            pltpu.async_copy(
                bufs[b], out_hbm.at[pl.ds(base + c * ch, ch)], ssems[b])

        def wait_store(b):
            pltpu.make_async_copy(
                bufs[b], out_hbm.at[pl.ds(base, ch)], ssems[b]).wait()

        pltpu.sync_copy(gidx_hbm.at[pl.ds(base, per_w)], idx_all)
        for b in range(3):
            fire_gather(b, b)

        nt = (nch - 2) // 3

        @pl.loop(0, nt)
        def _(p):
            c0 = 3 * p
            for b in range(3):
                wait_gather(b)
                fire_store(c0 + b, b)
            for b in range(3):
                wait_store(b)

                @pl.when(c0 + 3 + b < nch)
                def _():
                    fire_gather(c0 + 3 + b, b)

        for b in range(2):
            wait_gather(b)
            pltpu.sync_copy(bufs[b], out_hbm.at[pl.ds(base + (nch - 2 + b) * ch, ch)])

    return k(table, gidx)


def _tc_edge(g, conn, dstm2, ewt, eb2, sa, bbd, r1, r8, sb, prev):
    e = conn.shape[0]
    be = 2000
    gi = g.shape[0] // (3 * be)
    off = sb * gi

    def body(qd, ks, vs, cn, dm_r, ewt_r, eb_r, sa_r, bbd_r, r1_r, r8_r,
             eo, ua, ub, w2):
        eh = (
            jnp.dot(cn[...].astype(jnp.bfloat16), ewt_r[...],
                    preferred_element_type=jnp.float32)
            + eb_r[...]
        )
        ew = eh[:, :HD]
        ebb = eh[:, HD:]
        c1 = (_unpack(qd[...]) + _unpack(ks[...])) * ew
        c2 = jnp.sign(c1) * jnp.sqrt(jnp.abs(c1))
        co = jnp.maximum(c2 + ebb, 0.0)
        eo[...] = co
        cob = co.astype(jnp.bfloat16)
        s = jnp.clip(
            jnp.dot(cob, sa_r[...], preferred_element_type=jnp.float32), -5.0, 5.0
        )
        wb = jnp.exp(s).astype(jnp.bfloat16)
        wtile = jnp.dot(
            wb, r8_r[...], preferred_element_type=jnp.float32,
        )
        slot = lax.broadcasted_iota(jnp.int32, (be, 128), 1) // HEADS
        w2[...] = jnp.where(dm_r[...] == slot, wtile, 0.0)
        wrep = jnp.dot(
            wb, r1_r[...], preferred_element_type=jnp.float32,
        )
        u = (_unpack(vs[...])
             + jnp.dot(cob, bbd_r[...], preferred_element_type=jnp.float32)) * wrep
        ua[...] = u[:, :128]
        ub[...] = u[:, 128:]

    in_specs = [
        pl.BlockSpec((be, HIDDEN // 2), lambda i: (i, 0)),
        pl.BlockSpec((be, HIDDEN // 2), lambda i: (i + gi, 0)),
        pl.BlockSpec((be, HIDDEN // 2), lambda i: (i + 2 * gi, 0)),
        pl.BlockSpec((be, HIDDEN), lambda i: (i + off, 0)),
        pl.BlockSpec((be, 1), lambda i: (i + off, 0)),
        pl.BlockSpec((HIDDEN, 2 * HD), lambda i: (0, 0)),
        pl.BlockSpec((1, 2 * HD), lambda i: (0, 0)),
        pl.BlockSpec((HD, HEADS), lambda i: (0, 0)),
        pl.BlockSpec((HD, HD), lambda i: (0, 0)),
        pl.BlockSpec((HEADS, HD), lambda i: (0, 0)),
        pl.BlockSpec((HEADS, 128), lambda i: (0, 0)),
    ]
    args = [g, g, g, conn, dstm2, ewt, eb2, sa, bbd, r1, r8]
    aliases = {}
    if prev is not None:
        for j, a in enumerate(prev):
            in_specs.append(pl.BlockSpec(memory_space=pl.ANY))
            args.append(a)
            aliases[11 + j] = j

    def body_wrap(*refs):
        body(*refs[:11], *refs[-4:])

    return pl.pallas_call(
        body_wrap if prev is not None else body,
        grid=(gi,),
        in_specs=in_specs,
        out_specs=[
            pl.BlockSpec((be, HD), lambda i: (i + off, 0)),
            pl.BlockSpec((be, 128), lambda i: (i + off, 0)),
            pl.BlockSpec((be, 128), lambda i: (i + off, 0)),
            pl.BlockSpec((be, 128), lambda i: (i + off, 0)),
        ],
        out_shape=[
            jax.ShapeDtypeStruct((e, HD), jnp.float32),
            jax.ShapeDtypeStruct((e, 128), jnp.float32),
            jax.ShapeDtypeStruct((e, 128), jnp.float32),
            jax.ShapeDtypeStruct((e, 128), jnp.float32),
        ],
        input_output_aliases=aliases,
    )(*args)


def _sc_scatter(ua, ub, w2, dst, dst8):
    e = ua.shape[0]
    n = N_PAD
    nw = n // 8
    per_t = e // NS
    ch = 40
    nch = per_t // ch
    kq = 5
    nq = nch // kq
    rows_t = n // NS
    wrows_t = nw // NS

    mesh = plsc.VectorSubcoreMesh(core_axis_name="c", subcore_axis_name="s")

    @functools.partial(
        pl.kernel,
        out_type=(
            jax.ShapeDtypeStruct((n, 128), jnp.float32),
            jax.ShapeDtypeStruct((n, 128), jnp.float32),
            jax.ShapeDtypeStruct((2, nw, 128), jnp.float32),
        ),
        mesh=mesh,
        scratch_types=[
            pltpu.VMEM_SHARED((n, 128), jnp.float32),
            pltpu.VMEM_SHARED((nw, 128), jnp.float32),
            pltpu.VMEM((kq, ch), jnp.int32),
            pltpu.VMEM((kq, ch), jnp.int32),
            [pltpu.VMEM((ch, 128), jnp.float32) for _ in range(kq)],
            pltpu.SemaphoreType.DMA,
            pltpu.SemaphoreType.DMA,
        ],
    )
    def k(ua_h, ub_h, w2_h, dst3_h, dst83_h, out_a, out_b, out_w, acc_u, acc_w,
          idxq, idx8q, bufs, lsem, asem):
        cid = lax.axis_index("c")
        sid = lax.axis_index("s")

        @pl.loop(0, ch)
        def _(i):
            @pl.loop(0, 128 // 16)
            def _(j):
                bufs[0][i, pl.ds(j * 16, 16)] = jnp.zeros((16,), jnp.float32)

        r0 = sid * rows_t
        w0 = sid * wrows_t

        @pl.loop(0, rows_t // ch)
        def _(b):
            pltpu.sync_copy(bufs[0], acc_u.at[pl.ds(r0 + b * ch, ch)])

        @pl.loop(0, wrows_t // ch)
        def _(b):
            pltpu.sync_copy(bufs[0], acc_w.at[pl.ds(w0 + b * ch, ch)])

        plsc.subcore_barrier()

        base_e = sid * per_t

        def round_phase(src_h, acc, idx_h, idxb, p):
            base_c = p * kq
            pltpu.sync_copy(idx_h.at[sid * nq + p], idxb)
            for b in range(kq):
                pltpu.async_copy(
                    src_h.at[pl.ds(base_e + (base_c + b) * ch, ch)],
                    bufs[b], lsem)
            for b in range(kq):
                pltpu.make_async_copy(
                    src_h.at[pl.ds(base_e, ch)], bufs[b], lsem).wait()
            for b in range(kq):
                pltpu.async_copy(bufs[b], acc.at[idxb.at[b]], asem, add=True)
            for b in range(kq):
                pltpu.make_async_copy(bufs[b], acc.at[idxb.at[0]], asem).wait()

        def main_loop(src_h):
            @pl.loop(0, nq)
            def _(p):
                round_phase(src_h, acc_u, dst3_h, idxq, p)

                @pl.when(lax.rem(p, 2) == cid)
                def _():
                    round_phase(w2_h, acc_w, dst83_h, idx8q, p)

        @pl.when(cid == 0)
        def _():
            main_loop(ua_h)

        @pl.when(cid == 1)
        def _():
            main_loop(ub_h)

        plsc.subcore_barrier()

        @pl.when(cid == 0)
        def _():
            pltpu.sync_copy(acc_u.at[pl.ds(r0, rows_t)], out_a.at[pl.ds(r0, rows_t)])

        @pl.when(cid == 1)
        def _():
            pltpu.sync_copy(acc_u.at[pl.ds(r0, rows_t)], out_b.at[pl.ds(r0, rows_t)])

        pltpu.sync_copy(
            acc_w.at[pl.ds(w0, wrows_t)], out_w.at[cid].at[pl.ds(w0, wrows_t)]
        )

    dst3 = dst.reshape(NS * nq, kq, ch)
    dst83 = dst8.reshape(NS * nq, kq, ch)
    return k(ua, ub, w2, dst3, dst83)


def _tc_node(acc_a, acc_b, acc_w0, acc_w1, r1):
    n = N_NODES
    bn = 1000
    gi = n // bn

    def body(a, b, w0, w1, r1_r, o):
        den = (
            jnp.dot(
                w0[...] + w1[...], r1_r[...], preferred_element_type=jnp.float32,
                precision=lax.Precision.HIGHEST,
            )
            + 1e-16
        )
        o[...] = jnp.concatenate([a[...], b[...]], axis=1) / den

    return pl.pallas_call(
        body,
        grid=(gi,),
        in_specs=[
            pl.BlockSpec((bn, 128), lambda i: (i, 0)),
            pl.BlockSpec((bn, 128), lambda i: (i, 0)),
            pl.BlockSpec((bn, HEADS), lambda i: (i, 0)),
            pl.BlockSpec((bn, HEADS), lambda i: (i, 0)),
            pl.BlockSpec((HEADS, HD), lambda i: (0, 0)),
        ],
        out_specs=pl.BlockSpec((bn, HD), lambda i: (i, 0)),
        out_shape=jax.ShapeDtypeStruct((n, HD), jnp.float32),
    )(acc_a, acc_b, acc_w0, acc_w1, r1)


def kernel(x, rrwp_index, rrwp_conn, qkv_weight, qkv_bias, E_weight, E_bias, Aw, Bw):
    n = x.shape[0]
    dst = rrwp_index[0]
    src = rrwp_index[1]

    wt = qkv_weight.T
    b3 = qkv_bias.reshape(3, 1, HIDDEN)
    ewt = E_weight.T.astype(jnp.bfloat16)
    eb2 = E_bias.reshape(1, 2 * HD)
    eye = jnp.eye(HEADS, dtype=jnp.float32)
    sa = (Aw[:, :, 0].T[:, :, None] * eye[:, None, :]).reshape(HD, HEADS)
    sa = sa.astype(jnp.bfloat16)
    bbd = (Bw.transpose(1, 0, 2)[:, :, None, :] * eye[:, None, :, None]).reshape(HD, HD)
    bbd = bbd.astype(jnp.bfloat16)
    r1 = jnp.repeat(eye, ADIM, axis=1).astype(jnp.bfloat16)
    r8 = jnp.tile(eye, (1, 8)).astype(jnp.bfloat16)

    t32 = _tc_qkv(x, wt, b3)
    dstm2 = (dst % 8).reshape(-1, 1)
    dst8 = dst // 8

    gidx = jnp.concatenate([dst, src + n, src + 2 * n])
    g32 = _sc_gather(t32, gidx)
    eo, ua, ub, w2 = _tc_edge(g32, rrwp_conn, dstm2, ewt, eb2, sa, bbd, r1, r8,
                              0, None)
    acc_a, acc_b, acc_w = _sc_scatter(ua, ub, w2, dst, dst8)
    accw2 = acc_w.reshape(2, N_PAD, HEADS)
    no = _tc_node(acc_a, acc_b, accw2[0], accw2[1], jnp.repeat(eye, ADIM, axis=1))
    return no, eo


# speedup vs baseline: 1.2097x; 1.0676x over previous
import functools

import jax
import jax.numpy as jnp
from jax import lax
from jax.experimental import pallas as pl
from jax.experimental.pallas import tpu as pltpu
from jax.experimental.pallas import tpu_sc as plsc

def _f32_bits(x):
    return lax.bitcast_convert_type(
        x.astype(jnp.bfloat16).astype(jnp.float32), jnp.int32)


def _unpack_lo(p):
    return lax.bitcast_convert_type(lax.shift_left(p, 16), jnp.float32)


def _unpack_hi(p):
    return lax.bitcast_convert_type(p & jnp.int32(-65536), jnp.float32)


def _unpack(p):
    return jnp.concatenate([_unpack_lo(p), _unpack_hi(p)], axis=1)


N_NODES = 10000
N_EDGES = 160000
HIDDEN = 256
HEADS = 16
ADIM = 16
HD = HEADS * ADIM

NC = 2
NS = 16
N_PAD = 10240


def _tc_qkv(x, wt, b3):
    n = x.shape[0]
    bn = 1000
    gi = n // bn

    def body(x_ref, wt_ref, b_ref, o_ref):
        q = (
            jnp.dot(x_ref[...], wt_ref[...], preferred_element_type=jnp.float32)
            + b_ref[0]
        )
        a = _f32_bits(q[:, :128])
        b = _f32_bits(q[:, 128:])
        o_ref[...] = b | lax.shift_right_logical(a, 16)

    return pl.pallas_call(
        body,
        grid=(3, gi),
        in_specs=[
            pl.BlockSpec((bn, HIDDEN), lambda j, i: (i, 0)),
            pl.BlockSpec((HIDDEN, HIDDEN), lambda j, i: (0, j)),
            pl.BlockSpec((1, 1, HIDDEN), lambda j, i: (j, 0, 0)),
        ],
        out_specs=pl.BlockSpec((bn, HIDDEN // 2), lambda j, i: (j * gi + i, 0)),
        out_shape=jax.ShapeDtypeStruct((3 * n, HIDDEN // 2), jnp.int32),
    )(x, wt, b3)


def _sc_gather(table, gidx):
    rows = gidx.shape[0]
    width = table.shape[1]
    dt = table.dtype
    nw = NC * NS
    per_w = rows // nw
    ch = 120
    nch = per_w // ch

    mesh = plsc.VectorSubcoreMesh(core_axis_name="c", subcore_axis_name="s")

    @functools.partial(
        pl.kernel,
        out_type=jax.ShapeDtypeStruct((rows, width), dt),
        mesh=mesh,
        scratch_types=[
            pltpu.VMEM((per_w,), jnp.int32),
            pltpu.VMEM((ch, width), dt),
            pltpu.VMEM((ch, width), dt),
            pltpu.VMEM((ch, width), dt),
            pltpu.SemaphoreType.DMA,
            pltpu.SemaphoreType.DMA,
            pltpu.SemaphoreType.DMA,
            pltpu.SemaphoreType.DMA,
            pltpu.SemaphoreType.DMA,
            pltpu.SemaphoreType.DMA,
        ],
    )
    def k(table_hbm, gidx_hbm, out_hbm, idx_all, b0, b1, b2,
          gs0, gs1, gs2, ss0, ss1, ss2):
        cid = lax.axis_index("c")
        sid = lax.axis_index("s")
        base = (sid * NC + cid) * per_w

        bufs = (b0, b1, b2)
        gsems = (gs0, gs1, gs2)
        ssems = (ss0, ss1, ss2)

        def fire_gather(c, b):
            pltpu.async_copy(
                table_hbm.at[idx_all.at[pl.ds(c * ch, ch)]], bufs[b], gsems[b])

        def wait_gather(b):
            pltpu.make_async_copy(
                table_hbm.at[idx_all.at[pl.ds(0, ch)]], bufs[b], gsems[b]).wait()

        def fire_store(c, b):
            pltpu.async_copy(
                bufs[b], out_hbm.at[pl.ds(base + c * ch, ch)], ssems[b])

        def wait_store(b):
            pltpu.make_async_copy(
                bufs[b], out_hbm.at[pl.ds(base, ch)], ssems[b]).wait()

        pltpu.sync_copy(gidx_hbm.at[pl.ds(base, per_w)], idx_all)
        for b in range(3):
            fire_gather(b, b)

        nt = (nch - 2) // 3

        @pl.loop(0, nt)
        def _(p):
            c0 = 3 * p
            for b in range(3):
                wait_gather(b)
                fire_store(c0 + b, b)
            for b in range(3):
                wait_store(b)

                @pl.when(c0 + 3 + b < nch)
                def _():
                    fire_gather(c0 + 3 + b, b)

        for b in range(2):
            wait_gather(b)
            pltpu.sync_copy(bufs[b], out_hbm.at[pl.ds(base + (nch - 2 + b) * ch, ch)])

    return k(table, gidx)


def _tc_edge(g, conn, dstm2, ewt, eb2, sa, bbd, r1, r8, sb, prev):
    e = conn.shape[0]
    be = 2000
    gi = g.shape[0] // (3 * be)
    off = sb * gi

    def body(qd, ks, vs, cn, dm_r, ewt_r, eb_r, sa_r, bbd_r, r1_r, r8_r,
             eo, ua, ub, w2):
        eh = (
            jnp.dot(cn[...].astype(jnp.bfloat16), ewt_r[...],
                    preferred_element_type=jnp.float32)
            + eb_r[...]
        )
        ew = eh[:, :HD]
        ebb = eh[:, HD:]
        c1 = (_unpack(qd[...]) + _unpack(ks[...])) * ew
        c2 = jnp.sign(c1) * jnp.sqrt(jnp.abs(c1))
        co = jnp.maximum(c2 + ebb, 0.0)
        eo[...] = co
        cob = co.astype(jnp.bfloat16)
        s = jnp.clip(
            jnp.dot(cob, sa_r[...], preferred_element_type=jnp.float32), -5.0, 5.0
        )
        wb = jnp.exp(s).astype(jnp.bfloat16)
        wtile = jnp.dot(
            wb, r8_r[...], preferred_element_type=jnp.float32,
        )
        slot = lax.broadcasted_iota(jnp.int32, (be, 128), 1) // HEADS
        w2[...] = jnp.where(dm_r[...] == slot, wtile, 0.0)
        wrep = jnp.dot(
            wb, r1_r[...], preferred_element_type=jnp.float32,
        )
        u = (_unpack(vs[...])
             + jnp.dot(cob, bbd_r[...], preferred_element_type=jnp.float32)) * wrep
        ua[...] = u[:, :128]
        ub[...] = u[:, 128:]

    in_specs = [
        pl.BlockSpec((be, HIDDEN // 2), lambda i: (i, 0)),
        pl.BlockSpec((be, HIDDEN // 2), lambda i: (i + gi, 0)),
        pl.BlockSpec((be, HIDDEN // 2), lambda i: (i + 2 * gi, 0)),
        pl.BlockSpec((be, HIDDEN), lambda i: (i + off, 0)),
        pl.BlockSpec((be, 1), lambda i: (i + off, 0)),
        pl.BlockSpec((HIDDEN, 2 * HD), lambda i: (0, 0)),
        pl.BlockSpec((1, 2 * HD), lambda i: (0, 0)),
        pl.BlockSpec((HD, HEADS), lambda i: (0, 0)),
        pl.BlockSpec((HD, HD), lambda i: (0, 0)),
        pl.BlockSpec((HEADS, HD), lambda i: (0, 0)),
        pl.BlockSpec((HEADS, 128), lambda i: (0, 0)),
    ]
    args = [g, g, g, conn, dstm2, ewt, eb2, sa, bbd, r1, r8]
    aliases = {}
    if prev is not None:
        for j, a in enumerate(prev):
            in_specs.append(pl.BlockSpec(memory_space=pl.ANY))
            args.append(a)
            aliases[11 + j] = j

    def body_wrap(*refs):
        body(*refs[:11], *refs[-4:])

    return pl.pallas_call(
        body_wrap if prev is not None else body,
        grid=(gi,),
        in_specs=in_specs,
        out_specs=[
            pl.BlockSpec((be, HD), lambda i: (i + off, 0)),
            pl.BlockSpec((be, 128), lambda i: (i + off, 0)),
            pl.BlockSpec((be, 128), lambda i: (i + off, 0)),
            pl.BlockSpec((be, 128), lambda i: (i + off, 0)),
        ],
        out_shape=[
            jax.ShapeDtypeStruct((e, HD), jnp.float32),
            jax.ShapeDtypeStruct((e, 128), jnp.float32),
            jax.ShapeDtypeStruct((e, 128), jnp.float32),
            jax.ShapeDtypeStruct((e, 128), jnp.float32),
        ],
        input_output_aliases=aliases,
    )(*args)


def _sc_scatter(ua, ub, w2, dst, dst8):
    e = ua.shape[0]
    n = N_PAD
    nw = n // 8
    per_t = e // NS
    ch = 40
    nch = per_t // ch
    kq = 5
    nq = nch // kq
    rows_t = n // NS
    wrows_t = nw // NS

    mesh = plsc.VectorSubcoreMesh(core_axis_name="c", subcore_axis_name="s")

    @functools.partial(
        pl.kernel,
        out_type=(
            jax.ShapeDtypeStruct((n, 128), jnp.float32),
            jax.ShapeDtypeStruct((n, 128), jnp.float32),
            jax.ShapeDtypeStruct((2, nw, 128), jnp.float32),
        ),
        mesh=mesh,
        scratch_types=[
            pltpu.VMEM_SHARED((n, 128), jnp.float32),
            pltpu.VMEM_SHARED((nw, 128), jnp.float32),
            pltpu.VMEM((kq, ch), jnp.int32),
            pltpu.VMEM((kq, ch), jnp.int32),
            pltpu.VMEM((kq, ch), jnp.int32),
            [pltpu.VMEM((ch, 128), jnp.float32) for _ in range(kq)],
            [pltpu.SemaphoreType.DMA for _ in range(kq)],
            [pltpu.SemaphoreType.DMA for _ in range(kq)],
        ],
    )
    def k(ua_h, ub_h, w2_h, dst3_h, dst83_h, out_a, out_b, out_w, acc_u, acc_w,
          idxq0, idxq1, idx8q, bufs, lsem, asem):
        cid = lax.axis_index("c")
        sid = lax.axis_index("s")

        @pl.loop(0, ch)
        def _(i):
            @pl.loop(0, 128 // 16)
            def _(j):
                bufs[0][i, pl.ds(j * 16, 16)] = jnp.zeros((16,), jnp.float32)

        r0 = sid * rows_t
        w0 = sid * wrows_t

        @pl.loop(0, rows_t // ch)
        def _(b):
            pltpu.sync_copy(bufs[0], acc_u.at[pl.ds(r0 + b * ch, ch)])

        @pl.loop(0, wrows_t // ch)
        def _(b):
            pltpu.sync_copy(bufs[0], acc_w.at[pl.ds(w0 + b * ch, ch)])

        plsc.subcore_barrier()

        base_e = sid * per_t

        def fire_loads(src_h, p):
            for b in range(kq):
                pltpu.async_copy(
                    src_h.at[pl.ds(base_e + (p * kq + b) * ch, ch)],
                    bufs[b], lsem[b])

        def drain_load(src_h, b):
            pltpu.make_async_copy(
                src_h.at[pl.ds(base_e, ch)], bufs[b], lsem[b]).wait()

        def fire_add(acc, idxb, b):
            pltpu.async_copy(bufs[b], acc.at[idxb.at[b]], asem[b], add=True)

        def drain_add(b):
            pltpu.make_async_copy(bufs[b], acc_u.at[idxq0.at[0]], asem[b]).wait()

        def main_loop(src_h):
            pltpu.sync_copy(dst3_h.at[sid * nq], idxq0)
            fire_loads(src_h, 0)

            @pl.loop(0, nq // 2)
            def _(q):
                for half in range(2):
                    p = 2 * q + half
                    idx_cur = idxq0 if half == 0 else idxq1
                    idx_nxt = idxq1 if half == 0 else idxq0
                    for b in range(kq):
                        drain_load(src_h, b)
                        fire_add(acc_u, idx_cur, b)

                    @pl.when(cid == half)
                    def _():
                        pltpu.sync_copy(dst83_h.at[sid * nq + p], idx8q)
                        for b in range(kq):
                            drain_add(b)
                            pltpu.async_copy(
                                w2_h.at[pl.ds(base_e + (p * kq + b) * ch, ch)],
                                bufs[b], lsem[b])
                        for b in range(kq):
                            drain_load(w2_h, b)
                            fire_add(acc_w, idx8q, b)

                    @pl.when(p + 1 < nq)
                    def _():
                        pltpu.sync_copy(dst3_h.at[sid * nq + p + 1], idx_nxt)
                        for b in range(kq):
                            drain_add(b)
                            pltpu.async_copy(
                                src_h.at[pl.ds(base_e + ((p + 1) * kq + b) * ch, ch)],
                                bufs[b], lsem[b])

            for b in range(kq):
                drain_add(b)

        @pl.when(cid == 0)
        def _():
            main_loop(ua_h)

        @pl.when(cid == 1)
        def _():
            main_loop(ub_h)

        plsc.subcore_barrier()

        @pl.when(cid == 0)
        def _():
            pltpu.sync_copy(acc_u.at[pl.ds(r0, rows_t)], out_a.at[pl.ds(r0, rows_t)])

        @pl.when(cid == 1)
        def _():
            pltpu.sync_copy(acc_u.at[pl.ds(r0, rows_t)], out_b.at[pl.ds(r0, rows_t)])

        pltpu.sync_copy(
            acc_w.at[pl.ds(w0, wrows_t)], out_w.at[cid].at[pl.ds(w0, wrows_t)]
        )

    dst3 = dst.reshape(NS * nq, kq, ch)
    dst83 = dst8.reshape(NS * nq, kq, ch)
    return k(ua, ub, w2, dst3, dst83)


def _tc_node(acc_a, acc_b, acc_w0, acc_w1, r1):
    n = N_NODES
    bn = 1000
    gi = n // bn

    def body(a, b, w0, w1, r1_r, o):
        den = (
            jnp.dot(
                w0[...] + w1[...], r1_r[...], preferred_element_type=jnp.float32,
                precision=lax.Precision.HIGHEST,
            )
            + 1e-16
        )
        o[...] = jnp.concatenate([a[...], b[...]], axis=1) / den

    return pl.pallas_call(
        body,
        grid=(gi,),
        in_specs=[
            pl.BlockSpec((bn, 128), lambda i: (i, 0)),
            pl.BlockSpec((bn, 128), lambda i: (i, 0)),
            pl.BlockSpec((bn, HEADS), lambda i: (i, 0)),
            pl.BlockSpec((bn, HEADS), lambda i: (i, 0)),
            pl.BlockSpec((HEADS, HD), lambda i: (0, 0)),
        ],
        out_specs=pl.BlockSpec((bn, HD), lambda i: (i, 0)),
        out_shape=jax.ShapeDtypeStruct((n, HD), jnp.float32),
    )(acc_a, acc_b, acc_w0, acc_w1, r1)


def kernel(x, rrwp_index, rrwp_conn, qkv_weight, qkv_bias, E_weight, E_bias, Aw, Bw):
    n = x.shape[0]
    dst = rrwp_index[0]
    src = rrwp_index[1]

    wt = qkv_weight.T
    b3 = qkv_bias.reshape(3, 1, HIDDEN)
    ewt = E_weight.T.astype(jnp.bfloat16)
    eb2 = E_bias.reshape(1, 2 * HD)
    eye = jnp.eye(HEADS, dtype=jnp.float32)
    sa = (Aw[:, :, 0].T[:, :, None] * eye[:, None, :]).reshape(HD, HEADS)
    sa = sa.astype(jnp.bfloat16)
    bbd = (Bw.transpose(1, 0, 2)[:, :, None, :] * eye[:, None, :, None]).reshape(HD, HD)
    bbd = bbd.astype(jnp.bfloat16)
    r1 = jnp.repeat(eye, ADIM, axis=1).astype(jnp.bfloat16)
    r8 = jnp.tile(eye, (1, 8)).astype(jnp.bfloat16)

    t32 = _tc_qkv(x, wt, b3)
    dstm2 = (dst % 8).reshape(-1, 1)
    dst8 = dst // 8

    gidx = jnp.concatenate([dst, src + n, src + 2 * n])
    g32 = _sc_gather(t32, gidx)
    eo, ua, ub, w2 = _tc_edge(g32, rrwp_conn, dstm2, ewt, eb2, sa, bbd, r1, r8,
                              0, None)
    acc_a, acc_b, acc_w = _sc_scatter(ua, ub, w2, dst, dst8)
    accw2 = acc_w.reshape(2, N_PAD, HEADS)
    no = _tc_node(acc_a, acc_b, accw2[0], accw2[1], jnp.repeat(eye, ADIM, axis=1))
    return no, eo


# speedup vs baseline: 1.2251x; 1.0127x over previous
import functools

import jax
import jax.numpy as jnp
from jax import lax
from jax.experimental import pallas as pl
from jax.experimental.pallas import tpu as pltpu
from jax.experimental.pallas import tpu_sc as plsc

def _f32_bits(x):
    return lax.bitcast_convert_type(
        x.astype(jnp.bfloat16).astype(jnp.float32), jnp.int32)


def _unpack_lo(p):
    return lax.bitcast_convert_type(lax.shift_left(p, 16), jnp.float32)


def _unpack_hi(p):
    return lax.bitcast_convert_type(p & jnp.int32(-65536), jnp.float32)


def _unpack(p):
    return jnp.concatenate([_unpack_lo(p), _unpack_hi(p)], axis=1)


N_NODES = 10000
N_EDGES = 160000
HIDDEN = 256
HEADS = 16
ADIM = 16
HD = HEADS * ADIM

NC = 2
NS = 16
N_PAD = 10240


def _tc_qkv(x, wt, b3):
    n = x.shape[0]
    bn = 1000
    gi = n // bn

    def body(x_ref, wt_ref, b_ref, o_ref):
        q = (
            jnp.dot(x_ref[...], wt_ref[...], preferred_element_type=jnp.float32)
            + b_ref[0]
        )
        a = _f32_bits(q[:, :128])
        b = _f32_bits(q[:, 128:])
        o_ref[...] = b | lax.shift_right_logical(a, 16)

    return pl.pallas_call(
        body,
        grid=(3, gi),
        in_specs=[
            pl.BlockSpec((bn, HIDDEN), lambda j, i: (i, 0)),
            pl.BlockSpec((HIDDEN, HIDDEN), lambda j, i: (0, j)),
            pl.BlockSpec((1, 1, HIDDEN), lambda j, i: (j, 0, 0)),
        ],
        out_specs=pl.BlockSpec((bn, HIDDEN // 2), lambda j, i: (j * gi + i, 0)),
        out_shape=jax.ShapeDtypeStruct((3 * n, HIDDEN // 2), jnp.int32),
    )(x, wt, b3)


def _sc_gather(table, gidx):
    rows = gidx.shape[0]
    width = table.shape[1]
    dt = table.dtype
    nw = NC * NS
    per_w = rows // nw
    ch = 120
    nch = per_w // ch

    mesh = plsc.VectorSubcoreMesh(core_axis_name="c", subcore_axis_name="s")

    @functools.partial(
        pl.kernel,
        out_type=jax.ShapeDtypeStruct((rows, width), dt),
        mesh=mesh,
        scratch_types=[
            pltpu.VMEM((per_w,), jnp.int32),
            pltpu.VMEM((ch, width), dt),
            pltpu.VMEM((ch, width), dt),
            pltpu.VMEM((ch, width), dt),
            pltpu.SemaphoreType.DMA,
            pltpu.SemaphoreType.DMA,
            pltpu.SemaphoreType.DMA,
            pltpu.SemaphoreType.DMA,
            pltpu.SemaphoreType.DMA,
            pltpu.SemaphoreType.DMA,
        ],
    )
    def k(table_hbm, gidx_hbm, out_hbm, idx_all, b0, b1, b2,
          gs0, gs1, gs2, ss0, ss1, ss2):
        cid = lax.axis_index("c")
        sid = lax.axis_index("s")
        base = (sid * NC + cid) * per_w

        bufs = (b0, b1, b2)
        gsems = (gs0, gs1, gs2)
        ssems = (ss0, ss1, ss2)

        def fire_gather(c, b):
            pltpu.async_copy(
                table_hbm.at[idx_all.at[pl.ds(c * ch, ch)]], bufs[b], gsems[b])

        def wait_gather(b):
            pltpu.make_async_copy(
                table_hbm.at[idx_all.at[pl.ds(0, ch)]], bufs[b], gsems[b]).wait()

        def fire_store(c, b):
            pltpu.async_copy(
                bufs[b], out_hbm.at[pl.ds(base + c * ch, ch)], ssems[b])

        def wait_store(b):
            pltpu.make_async_copy(
                bufs[b], out_hbm.at[pl.ds(base, ch)], ssems[b]).wait()

        pltpu.sync_copy(gidx_hbm.at[pl.ds(base, per_w)], idx_all)
        for b in range(3):
            fire_gather(b, b)

        nt = (nch - 2) // 3

        @pl.loop(0, nt)
        def _(p):
            c0 = 3 * p
            for b in range(3):
                wait_gather(b)
                fire_store(c0 + b, b)
            for b in range(3):
                wait_store(b)

                @pl.when(c0 + 3 + b < nch)
                def _():
                    fire_gather(c0 + 3 + b, b)

        for b in range(2):
            wait_gather(b)
            pltpu.sync_copy(bufs[b], out_hbm.at[pl.ds(base + (nch - 2 + b) * ch, ch)])

    return k(table, gidx)


def _tc_edge(g, conn, dstm2, ewt, eb2, sa, bbd, r1, r8, sb, prev):
    e = conn.shape[0]
    be = 4000
    gi = g.shape[0] // (3 * be)
    off = sb * gi

    def body(qd, ks, vs, cn, dm_r, ewt_r, eb_r, sa_r, bbd_r, r1_r, r8_r,
             eo, ua, ub, w2):
        eh = (
            jnp.dot(cn[...].astype(jnp.bfloat16), ewt_r[...],
                    preferred_element_type=jnp.float32)
            + eb_r[...]
        )
        ew = eh[:, :HD]
        ebb = eh[:, HD:]
        c1 = (_unpack(qd[...]) + _unpack(ks[...])) * ew
        c2 = jnp.sign(c1) * jnp.sqrt(jnp.abs(c1))
        co = jnp.maximum(c2 + ebb, 0.0)
        eo[...] = co
        cob = co.astype(jnp.bfloat16)
        s = jnp.clip(
            jnp.dot(cob, sa_r[...], preferred_element_type=jnp.float32), -5.0, 5.0
        )
        wb = jnp.exp(s).astype(jnp.bfloat16)
        wtile = jnp.dot(
            wb, r8_r[...], preferred_element_type=jnp.float32,
        )
        slot = lax.broadcasted_iota(jnp.int32, (be, 128), 1) // HEADS
        w2[...] = jnp.where(dm_r[...] == slot, wtile, 0.0)
        wrep = jnp.dot(
            wb, r1_r[...], preferred_element_type=jnp.float32,
        )
        u = (_unpack(vs[...])
             + jnp.dot(cob, bbd_r[...], preferred_element_type=jnp.float32)) * wrep
        ua[...] = u[:, :128]
        ub[...] = u[:, 128:]

    in_specs = [
        pl.BlockSpec((be, HIDDEN // 2), lambda i: (i, 0)),
        pl.BlockSpec((be, HIDDEN // 2), lambda i: (i + gi, 0)),
        pl.BlockSpec((be, HIDDEN // 2), lambda i: (i + 2 * gi, 0)),
        pl.BlockSpec((be, HIDDEN), lambda i: (i + off, 0)),
        pl.BlockSpec((be, 1), lambda i: (i + off, 0)),
        pl.BlockSpec((HIDDEN, 2 * HD), lambda i: (0, 0)),
        pl.BlockSpec((1, 2 * HD), lambda i: (0, 0)),
        pl.BlockSpec((HD, HEADS), lambda i: (0, 0)),
        pl.BlockSpec((HD, HD), lambda i: (0, 0)),
        pl.BlockSpec((HEADS, HD), lambda i: (0, 0)),
        pl.BlockSpec((HEADS, 128), lambda i: (0, 0)),
    ]
    args = [g, g, g, conn, dstm2, ewt, eb2, sa, bbd, r1, r8]
    aliases = {}
    if prev is not None:
        for j, a in enumerate(prev):
            in_specs.append(pl.BlockSpec(memory_space=pl.ANY))
            args.append(a)
            aliases[11 + j] = j

    def body_wrap(*refs):
        body(*refs[:11], *refs[-4:])

    return pl.pallas_call(
        body_wrap if prev is not None else body,
        grid=(gi,),
        in_specs=in_specs,
        out_specs=[
            pl.BlockSpec((be, HD), lambda i: (i + off, 0)),
            pl.BlockSpec((be, 128), lambda i: (i + off, 0)),
            pl.BlockSpec((be, 128), lambda i: (i + off, 0)),
            pl.BlockSpec((be, 128), lambda i: (i + off, 0)),
        ],
        out_shape=[
            jax.ShapeDtypeStruct((e, HD), jnp.float32),
            jax.ShapeDtypeStruct((e, 128), jnp.float32),
            jax.ShapeDtypeStruct((e, 128), jnp.float32),
            jax.ShapeDtypeStruct((e, 128), jnp.float32),
        ],
        input_output_aliases=aliases,
    )(*args)


def _sc_scatter(ua, ub, w2, dst, dst8):
    e = ua.shape[0]
    n = N_PAD
    nw = n // 8
    per_t = e // NS
    ch = 40
    nch = per_t // ch
    kq = 5
    nq = nch // kq
    rows_t = n // NS
    wrows_t = nw // NS

    mesh = plsc.VectorSubcoreMesh(core_axis_name="c", subcore_axis_name="s")

    @functools.partial(
        pl.kernel,
        out_type=(
            jax.ShapeDtypeStruct((n, 128), jnp.float32),
            jax.ShapeDtypeStruct((n, 128), jnp.float32),
            jax.ShapeDtypeStruct((2, nw, 128), jnp.float32),
        ),
        mesh=mesh,
        scratch_types=[
            pltpu.VMEM_SHARED((n, 128), jnp.float32),
            pltpu.VMEM_SHARED((nw, 128), jnp.float32),
            pltpu.VMEM((kq, ch), jnp.int32),
            pltpu.VMEM((kq, ch), jnp.int32),
            pltpu.VMEM((kq, ch), jnp.int32),
            [pltpu.VMEM((ch, 128), jnp.float32) for _ in range(kq)],
            [pltpu.SemaphoreType.DMA for _ in range(kq)],
            [pltpu.SemaphoreType.DMA for _ in range(kq)],
        ],
    )
    def k(ua_h, ub_h, w2_h, dst3_h, dst83_h, out_a, out_b, out_w, acc_u, acc_w,
          idxq0, idxq1, idx8q, bufs, lsem, asem):
        cid = lax.axis_index("c")
        sid = lax.axis_index("s")

        @pl.loop(0, ch)
        def _(i):
            @pl.loop(0, 128 // 16)
            def _(j):
                bufs[0][i, pl.ds(j * 16, 16)] = jnp.zeros((16,), jnp.float32)

        r0 = sid * rows_t
        w0 = sid * wrows_t

        @pl.loop(0, rows_t // ch)
        def _(b):
            pltpu.sync_copy(bufs[0], acc_u.at[pl.ds(r0 + b * ch, ch)])

        @pl.loop(0, wrows_t // ch)
        def _(b):
            pltpu.sync_copy(bufs[0], acc_w.at[pl.ds(w0 + b * ch, ch)])

        plsc.subcore_barrier()

        base_e = sid * per_t

        def fire_loads(src_h, p):
            for b in range(kq):
                pltpu.async_copy(
                    src_h.at[pl.ds(base_e + (p * kq + b) * ch, ch)],
                    bufs[b], lsem[b])

        def drain_load(src_h, b):
            pltpu.make_async_copy(
                src_h.at[pl.ds(base_e, ch)], bufs[b], lsem[b]).wait()

        def fire_add(acc, idxb, b):
            pltpu.async_copy(bufs[b], acc.at[idxb.at[b]], asem[b], add=True)

        def drain_add(b):
            pltpu.make_async_copy(bufs[b], acc_u.at[idxq0.at[0]], asem[b]).wait()

        def main_loop(src_h):
            pltpu.sync_copy(dst3_h.at[sid * nq], idxq0)
            fire_loads(src_h, 0)

            @pl.loop(0, nq // 2)
            def _(q):
                for half in range(2):
                    p = 2 * q + half
                    idx_cur = idxq0 if half == 0 else idxq1
                    idx_nxt = idxq1 if half == 0 else idxq0
                    for b in range(kq):
                        drain_load(src_h, b)
                        fire_add(acc_u, idx_cur, b)

                    @pl.when(cid == half)
                    def _():
                        pltpu.sync_copy(dst83_h.at[sid * nq + p], idx8q)
                        for b in range(kq):
                            drain_add(b)
                            pltpu.async_copy(
                                w2_h.at[pl.ds(base_e + (p * kq + b) * ch, ch)],
                                bufs[b], lsem[b])
                        for b in range(kq):
                            drain_load(w2_h, b)
                            fire_add(acc_w, idx8q, b)

                    @pl.when(p + 1 < nq)
                    def _():
                        pltpu.sync_copy(dst3_h.at[sid * nq + p + 1], idx_nxt)
                        for b in range(kq):
                            drain_add(b)
                            pltpu.async_copy(
                                src_h.at[pl.ds(base_e + ((p + 1) * kq + b) * ch, ch)],
                                bufs[b], lsem[b])

            for b in range(kq):
                drain_add(b)

        @pl.when(cid == 0)
        def _():
            main_loop(ua_h)

        @pl.when(cid == 1)
        def _():
            main_loop(ub_h)

        plsc.subcore_barrier()

        @pl.when(cid == 0)
        def _():
            pltpu.sync_copy(acc_u.at[pl.ds(r0, rows_t)], out_a.at[pl.ds(r0, rows_t)])

        @pl.when(cid == 1)
        def _():
            pltpu.sync_copy(acc_u.at[pl.ds(r0, rows_t)], out_b.at[pl.ds(r0, rows_t)])

        pltpu.sync_copy(
            acc_w.at[pl.ds(w0, wrows_t)], out_w.at[cid].at[pl.ds(w0, wrows_t)]
        )

    dst3 = dst.reshape(NS * nq, kq, ch)
    dst83 = dst8.reshape(NS * nq, kq, ch)
    return k(ua, ub, w2, dst3, dst83)


def _tc_node(acc_a, acc_b, acc_w0, acc_w1, r1):
    n = N_NODES
    bn = 1000
    gi = n // bn

    def body(a, b, w0, w1, r1_r, o):
        den = (
            jnp.dot(
                w0[...] + w1[...], r1_r[...], preferred_element_type=jnp.float32,
                precision=lax.Precision.HIGHEST,
            )
            + 1e-16
        )
        o[...] = jnp.concatenate([a[...], b[...]], axis=1) / den

    return pl.pallas_call(
        body,
        grid=(gi,),
        in_specs=[
            pl.BlockSpec((bn, 128), lambda i: (i, 0)),
            pl.BlockSpec((bn, 128), lambda i: (i, 0)),
            pl.BlockSpec((bn, HEADS), lambda i: (i, 0)),
            pl.BlockSpec((bn, HEADS), lambda i: (i, 0)),
            pl.BlockSpec((HEADS, HD), lambda i: (0, 0)),
        ],
        out_specs=pl.BlockSpec((bn, HD), lambda i: (i, 0)),
        out_shape=jax.ShapeDtypeStruct((n, HD), jnp.float32),
    )(acc_a, acc_b, acc_w0, acc_w1, r1)


def kernel(x, rrwp_index, rrwp_conn, qkv_weight, qkv_bias, E_weight, E_bias, Aw, Bw):
    n = x.shape[0]
    dst = rrwp_index[0]
    src = rrwp_index[1]

    wt = qkv_weight.T
    b3 = qkv_bias.reshape(3, 1, HIDDEN)
    ewt = E_weight.T.astype(jnp.bfloat16)
    eb2 = E_bias.reshape(1, 2 * HD)
    eye = jnp.eye(HEADS, dtype=jnp.float32)
    sa = (Aw[:, :, 0].T[:, :, None] * eye[:, None, :]).reshape(HD, HEADS)
    sa = sa.astype(jnp.bfloat16)
    bbd = (Bw.transpose(1, 0, 2)[:, :, None, :] * eye[:, None, :, None]).reshape(HD, HD)
    bbd = bbd.astype(jnp.bfloat16)
    r1 = jnp.repeat(eye, ADIM, axis=1).astype(jnp.bfloat16)
    r8 = jnp.tile(eye, (1, 8)).astype(jnp.bfloat16)

    t32 = _tc_qkv(x, wt, b3)
    dstm2 = (dst % 8).reshape(-1, 1)
    dst8 = dst // 8

    gidx = jnp.concatenate([dst, src + n, src + 2 * n])
    g32 = _sc_gather(t32, gidx)
    eo, ua, ub, w2 = _tc_edge(g32, rrwp_conn, dstm2, ewt, eb2, sa, bbd, r1, r8,
                              0, None)
    acc_a, acc_b, acc_w = _sc_scatter(ua, ub, w2, dst, dst8)
    accw2 = acc_w.reshape(2, N_PAD, HEADS)
    no = _tc_node(acc_a, acc_b, accw2[0], accw2[1], jnp.repeat(eye, ADIM, axis=1))
    return no, eo
